# Initial kernel scaffold; baseline (speedup 1.0000x reference)
import functools

import jax
import jax.numpy as jnp
from jax import lax
from jax.experimental import pallas as pl
from jax.experimental.pallas import tpu as pltpu
from jax.experimental.pallas import tpu_sc as plsc

N_FIELDS = 26
VOCAB = 100000
D = 16
BATCH = 16384
N_CONT = 13

NC = 2
NS = 16
NW = NC * NS
DISC_ROWS = N_FIELDS * BATCH
RPW = DISC_ROWS // NW
G = 128
NG = RPW // G
SUPER = 1024
NSUP = RPW // SUPER
GPS = SUPER // G
CB = BATCH // NW
CONT_BASE = DISC_ROWS
TOTAL_ROWS = (N_FIELDS + N_CONT) * BATCH


def _sc_body(tab, idxh, xch, cwh, outh,
             idx_v, rows_v, xc_v, cw_v, cont_v, gsem, wsem, csem):
    wid = lax.axis_index("s") * NC + lax.axis_index("c")
    base_row = wid * RPW

    pltpu.sync_copy(idxh.at[wid], idx_v)
    pltpu.sync_copy(xch.at[wid], xc_v)
    pltpu.sync_copy(cwh, cw_v)

    lane = lax.iota(jnp.int32, 16)

    def adjust(g, carry):
        for k in range(G // 16):
            r = base_row + g * G + (k * 16) + lane
            f = jnp.right_shift(r, 14)
            idx_v[g, pl.ds(k * 16, 16)] = idx_v[g, pl.ds(k * 16, 16)] + f * VOCAB
        return carry

    lax.fori_loop(0, NG, adjust, 0)

    for s in range(NSUP):
        p = s % 2
        if s >= 2:
            pltpu.make_async_copy(
                rows_v.at[p],
                outh.at[pl.ds(base_row + (s - 2) * SUPER, SUPER)], wsem).wait()
            pltpu.make_async_copy(
                cont_v.at[p], outh.at[pl.ds(0, CB)], csem).wait()
        for g8 in range(GPS):
            g = s * GPS + g8
            pltpu.async_copy(tab.at[idx_v.at[g]],
                             rows_v.at[p, pl.ds(g8 * G, G)], gsem)
        cw = cw_v[s]

        def cont_row(b, carry):
            cont_v[p, b, :] = cw * xc_v[s, b]
            return carry

        lax.fori_loop(0, CB, cont_row, 0)
        pltpu.async_copy(
            cont_v.at[p],
            outh.at[pl.ds(CONT_BASE + s * BATCH + wid * CB, CB)], csem)
        pltpu.make_async_copy(tab.at[pl.ds(0, SUPER)], rows_v.at[p], gsem).wait()
        pltpu.async_copy(rows_v.at[p],
                         outh.at[pl.ds(base_row + s * SUPER, SUPER)], wsem)

    for s in (NSUP - 2, NSUP - 1):
        p = s % 2
        pltpu.make_async_copy(
            rows_v.at[p],
            outh.at[pl.ds(base_row + s * SUPER, SUPER)], wsem).wait()
        pltpu.make_async_copy(cont_v.at[p], outh.at[pl.ds(0, CB)], csem).wait()


_sc_call = pl.kernel(
    _sc_body,
    out_type=jax.ShapeDtypeStruct((TOTAL_ROWS, D), jnp.float32),
    mesh=plsc.VectorSubcoreMesh(core_axis_name="c", subcore_axis_name="s"),
    scratch_types=[
        pltpu.VMEM((NG, G), jnp.int32),
        pltpu.VMEM((2, SUPER, D), jnp.float32),
        pltpu.VMEM((N_CONT, CB), jnp.float32),
        pltpu.VMEM((N_CONT, D), jnp.float32),
        pltpu.VMEM((2, CB, D), jnp.float32),
        pltpu.SemaphoreType.DMA,
        pltpu.SemaphoreType.DMA,
        pltpu.SemaphoreType.DMA,
    ],
)


@jax.jit
def kernel(x_disc, x_cont, tables, cont_w):
    idx = x_disc.astype(jnp.int32).T.reshape(NW, NG, G)
    xc = x_cont.reshape(NW, CB, N_CONT).transpose(0, 2, 1)
    tab = tables.reshape(N_FIELDS * VOCAB, D)
    out = _sc_call(tab, idx, xc, cont_w)
    return out.reshape(N_FIELDS + N_CONT, BATCH, D)


# speedup vs baseline: 1.0854x; 1.0854x over previous
import functools

import jax
import jax.numpy as jnp
from jax import lax
from jax.experimental import pallas as pl
from jax.experimental.pallas import tpu as pltpu
from jax.experimental.pallas import tpu_sc as plsc

N_FIELDS = 26
VOCAB = 100000
D = 16
BATCH = 16384
N_CONT = 13

NC = 2
NS = 16
NW = NC * NS
DISC_ROWS = N_FIELDS * BATCH
RPW = DISC_ROWS // NW
G = 128
NG = RPW // G
SUPER = 1024
NSUP = RPW // SUPER
GPS = SUPER // G
CB = BATCH // NW
CONT_BASE = DISC_ROWS
TOTAL_ROWS = (N_FIELDS + N_CONT) * BATCH


def _sc_body(tab, idxh, xch, cwh, outh,
             idx_v, rows_v, xc_v, cw_v, cont_v, gsem, wsem, csem):
    wid = lax.axis_index("s") * NC + lax.axis_index("c")
    base_row = wid * RPW

    pltpu.sync_copy(idxh.at[wid], idx_v)
    pltpu.sync_copy(xch.at[wid], xc_v)
    pltpu.sync_copy(cwh, cw_v)

    lane = lax.iota(jnp.int32, 16)

    def adjust(g, carry):
        for k in range(G // 16):
            r = base_row + g * G + (k * 16) + lane
            f = jnp.right_shift(r, 14)
            idx_v[g, pl.ds(k * 16, 16)] = idx_v[g, pl.ds(k * 16, 16)] + f * VOCAB
        return carry

    lax.fori_loop(0, NG, adjust, 0)

    for s in range(NSUP):
        p = s % 2
        if s >= 2:
            pltpu.make_async_copy(
                rows_v.at[p],
                outh.at[pl.ds(base_row + (s - 2) * SUPER, SUPER)], wsem).wait()
            pltpu.make_async_copy(
                cont_v.at[p], outh.at[pl.ds(0, CB)], csem).wait()
        for g8 in range(GPS):
            g = s * GPS + g8
            pltpu.async_copy(tab.at[idx_v.at[g]],
                             rows_v.at[p, pl.ds(g8 * G, G)], gsem)
        cw = cw_v[s]

        def cont_row(b16, carry):
            xcv = xc_v[s, pl.ds(b16 * 16, 16)]
            for l in range(16):
                cont_v[p, b16 * 16 + l, :] = cw * xcv[l]
            return carry

        lax.fori_loop(0, CB // 16, cont_row, 0)
        pltpu.async_copy(
            cont_v.at[p],
            outh.at[pl.ds(CONT_BASE + s * BATCH + wid * CB, CB)], csem)
        pltpu.make_async_copy(tab.at[pl.ds(0, SUPER)], rows_v.at[p], gsem).wait()
        pltpu.async_copy(rows_v.at[p],
                         outh.at[pl.ds(base_row + s * SUPER, SUPER)], wsem)

    for s in (NSUP - 2, NSUP - 1):
        p = s % 2
        pltpu.make_async_copy(
            rows_v.at[p],
            outh.at[pl.ds(base_row + s * SUPER, SUPER)], wsem).wait()
        pltpu.make_async_copy(cont_v.at[p], outh.at[pl.ds(0, CB)], csem).wait()


_sc_call = pl.kernel(
    _sc_body,
    out_type=jax.ShapeDtypeStruct((TOTAL_ROWS, D), jnp.float32),
    mesh=plsc.VectorSubcoreMesh(core_axis_name="c", subcore_axis_name="s"),
    compiler_params=pltpu.CompilerParams(use_tc_tiling_on_sc=False),
    scratch_types=[
        pltpu.VMEM((NG, G), jnp.int32),
        pltpu.VMEM((2, SUPER, D), jnp.float32),
        pltpu.VMEM((N_CONT, CB), jnp.float32),
        pltpu.VMEM((N_CONT, D), jnp.float32),
        pltpu.VMEM((2, CB, D), jnp.float32),
        pltpu.SemaphoreType.DMA,
        pltpu.SemaphoreType.DMA,
        pltpu.SemaphoreType.DMA,
    ],
)


@jax.jit
def kernel(x_disc, x_cont, tables, cont_w):
    idx = x_disc.astype(jnp.int32).T.reshape(NW, NG, G)
    xc = x_cont.reshape(NW, CB, N_CONT).transpose(0, 2, 1)
    tab = tables.reshape(N_FIELDS * VOCAB, D)
    out = _sc_call(tab, idx, xc, cont_w)
    return out.reshape(N_FIELDS + N_CONT, BATCH, D)


# speedup vs baseline: 5.9216x; 5.4554x over previous
import functools

import jax
import jax.numpy as jnp
from jax import lax
from jax.experimental import pallas as pl
from jax.experimental.pallas import tpu as pltpu
from jax.experimental.pallas import tpu_sc as plsc

N_FIELDS = 26
VOCAB = 100000
D = 16
BATCH = 16384
N_CONT = 13

NC = 2
NS = 16
NW = NC * NS
DISC_STRIPS = N_FIELDS * D
CONT_STRIPS = N_CONT * D
DSPW = DISC_STRIPS // NW
CSPW = -(-CONT_STRIPS // NW)
HALF = BATCH // 2


def _sc_body(tab_t, xd_t, xc_t, cwf, outh, row_v, idx_v, out_v, cw_v):
    wid = lax.axis_index("s") * NC + lax.axis_index("c")
    pltpu.sync_copy(cwf, cw_v)

    for t in range(DSPW):
        strip = wid * DSPW + t
        f = jnp.right_shift(strip, 4)
        d = jnp.bitwise_and(strip, 15)
        pltpu.sync_copy(tab_t.at[f, d], row_v)
        for h in range(2):
            pltpu.sync_copy(xd_t.at[f, pl.ds(h * HALF, HALF)], idx_v)

            def gk(k, carry):
                iv = idx_v[pl.ds(k * 16, 16)]
                out_v[pl.ds(k * 16, 16)] = plsc.load_gather(row_v, [iv])
                return carry

            lax.fori_loop(0, HALF // 16, gk, 0)
            pltpu.sync_copy(out_v, outh.at[f, d, pl.ds(h * HALF, HALF)])

    for t in range(CSPW):
        q = wid * CSPW + t

        @pl.when(q < CONT_STRIPS)
        def _():
            j = jnp.right_shift(q, 4)
            d = jnp.bitwise_and(q, 15)
            cws = plsc.load_gather(cw_v, [jnp.broadcast_to(q, (16,))])
            for h in range(2):
                pltpu.sync_copy(xc_t.at[j, pl.ds(h * HALF, HALF)],
                                row_v.at[pl.ds(0, HALF)])

                def ck(k, carry):
                    out_v[pl.ds(k * 16, 16)] = cws * row_v[pl.ds(k * 16, 16)]
                    return carry

                lax.fori_loop(0, HALF // 16, ck, 0)
                pltpu.sync_copy(out_v,
                                outh.at[N_FIELDS + j, d, pl.ds(h * HALF, HALF)])


_sc_call = pl.kernel(
    _sc_body,
    out_type=jax.ShapeDtypeStruct((N_FIELDS + N_CONT, D, BATCH), jnp.float32),
    mesh=plsc.VectorSubcoreMesh(core_axis_name="c", subcore_axis_name="s"),
    compiler_params=pltpu.CompilerParams(use_tc_tiling_on_sc=True,
                                         needs_layout_passes=False),
    scratch_types=[
        pltpu.VMEM((VOCAB,), jnp.float32),
        pltpu.VMEM((HALF,), jnp.int32),
        pltpu.VMEM((HALF,), jnp.float32),
        pltpu.VMEM((CONT_STRIPS,), jnp.float32),
    ],
)


@jax.jit
def kernel(x_disc, x_cont, tables, cont_w):
    tab_t = tables.transpose(0, 2, 1)
    xd_t = x_disc.astype(jnp.int32).T
    xc_t = x_cont.T
    cwf = cont_w.reshape(CONT_STRIPS)
    out_t = _sc_call(tab_t, xd_t, xc_t, cwf)
    return out_t.transpose(0, 2, 1)


# speedup vs baseline: 6.2549x; 1.0563x over previous
import functools

import jax
import jax.numpy as jnp
from jax import lax
from jax.experimental import pallas as pl
from jax.experimental.pallas import tpu as pltpu
from jax.experimental.pallas import tpu_sc as plsc

N_FIELDS = 26
VOCAB = 100000
D = 16
BATCH = 16384
N_CONT = 13

NC = 2
NS = 16
NW = NC * NS
DISC_STRIPS = N_FIELDS * D
CONT_STRIPS = N_CONT * D
DSPW = DISC_STRIPS // NW
CSPW = -(-CONT_STRIPS // NW)
HALF = BATCH // 2


def _sc_body(tab_t, xd_t, xc_t, cwf, outh, row_v, idx_v, out_v, cw_v, rsem, wsem):
    wid = lax.axis_index("s") * NC + lax.axis_index("c")
    pltpu.sync_copy(cwf, cw_v)

    def wait_out_write():
        pltpu.make_async_copy(out_v, outh.at[0, 0, pl.ds(0, HALF)], wsem).wait()

    first_write = [True]

    def reclaim_out():
        if first_write[0]:
            first_write[0] = False
        else:
            wait_out_write()

    for t in range(DSPW):
        strip = wid * DSPW + t
        f = jnp.right_shift(strip, 4)
        d = jnp.bitwise_and(strip, 15)
        row_cp = pltpu.async_copy(tab_t.at[f, d], row_v, rsem)
        if t == 0:
            pltpu.sync_copy(xd_t.at[f], idx_v)
        else:
            prev_f = jnp.right_shift(strip - 1, 4)

            @pl.when(f != prev_f)
            def _():
                pltpu.sync_copy(xd_t.at[f], idx_v)

        row_cp.wait()
        for h in range(2):
            reclaim_out()

            def gk(k, carry):
                iv = idx_v[pl.ds(h * HALF + k * 16, 16)]
                out_v[pl.ds(k * 16, 16)] = plsc.load_gather(row_v, [iv])
                return carry

            lax.fori_loop(0, HALF // 16, gk, 0, unroll=8)
            pltpu.async_copy(out_v, outh.at[f, d, pl.ds(h * HALF, HALF)], wsem)

    for t in range(CSPW):
        q = wid * CSPW + t

        @pl.when(q < CONT_STRIPS)
        def _():
            j = jnp.right_shift(q, 4)
            d = jnp.bitwise_and(q, 15)
            cws = plsc.load_gather(cw_v, [jnp.broadcast_to(q, (16,))])
            for h in range(2):
                wait_out_write()
                pltpu.sync_copy(xc_t.at[j, pl.ds(h * HALF, HALF)], out_v)

                def ck(k, carry):
                    out_v[pl.ds(k * 16, 16)] = cws * out_v[pl.ds(k * 16, 16)]
                    return carry

                lax.fori_loop(0, HALF // 16, ck, 0, unroll=8)
                pltpu.async_copy(out_v,
                                 outh.at[N_FIELDS + j, d, pl.ds(h * HALF, HALF)],
                                 wsem)

    wait_out_write()


_sc_call = pl.kernel(
    _sc_body,
    out_type=jax.ShapeDtypeStruct((N_FIELDS + N_CONT, D, BATCH), jnp.float32),
    mesh=plsc.VectorSubcoreMesh(core_axis_name="c", subcore_axis_name="s"),
    compiler_params=pltpu.CompilerParams(use_tc_tiling_on_sc=True,
                                         needs_layout_passes=False),
    scratch_types=[
        pltpu.VMEM((VOCAB,), jnp.float32),
        pltpu.VMEM((BATCH,), jnp.int32),
        pltpu.VMEM((HALF,), jnp.float32),
        pltpu.VMEM((CONT_STRIPS,), jnp.float32),
        pltpu.SemaphoreType.DMA,
        pltpu.SemaphoreType.DMA,
    ],
)


@jax.jit
def kernel(x_disc, x_cont, tables, cont_w):
    tab_t = tables.transpose(0, 2, 1)
    xd_t = x_disc.astype(jnp.int32).T
    xc_t = x_cont.T
    cwf = cont_w.reshape(CONT_STRIPS)
    out_t = _sc_call(tab_t, xd_t, xc_t, cwf)
    return out_t.transpose(0, 2, 1)


# speedup vs baseline: 11.1076x; 1.7758x over previous
import functools

import jax
import jax.numpy as jnp
from jax import lax
from jax.experimental import pallas as pl
from jax.experimental.pallas import tpu as pltpu
from jax.experimental.pallas import tpu_sc as plsc

N_FIELDS = 26
VOCAB = 100000
D = 16
BATCH = 16384
N_CONT = 13

NC = 2
NS = 16
NW = NC * NS
DISC_STRIPS = N_FIELDS * D
CONT_STRIPS = N_CONT * D
DSPW = DISC_STRIPS // NW
CSPW = -(-CONT_STRIPS // NW)
HALF = BATCH // 2


def _sc_body(tab_t, xd_t, xc_t, cwf, outh, row_v, idx_v, out_v, cw_v, rsem, wsem):
    wid = lax.axis_index("s") * NC + lax.axis_index("c")
    pltpu.sync_copy(cwf, cw_v)

    def wait_out_write():
        pltpu.make_async_copy(out_v, outh.at[0, 0, pl.ds(0, HALF)], wsem).wait()

    def cont_strip(ct):
        q = wid * CSPW + ct

        @pl.when(q < CONT_STRIPS)
        def _():
            j = jnp.right_shift(q, 4)
            d = jnp.bitwise_and(q, 15)
            cws = plsc.load_gather(cw_v, [jnp.broadcast_to(q, (16,))])
            for h in range(2):
                wait_out_write()
                pltpu.sync_copy(xc_t.at[j, pl.ds(h * HALF, HALF)], out_v)

                def ck(k, carry):
                    vs = [out_v[pl.ds(k * 128 + i * 16, 16)] for i in range(8)]
                    prods = [cws * v for v in vs]
                    for i in range(8):
                        out_v[pl.ds(k * 128 + i * 16, 16)] = prods[i]
                    return carry

                lax.fori_loop(0, HALF // 128, ck, 0, unroll=2)
                pltpu.async_copy(out_v,
                                 outh.at[N_FIELDS + j, d, pl.ds(h * HALF, HALF)],
                                 wsem)

    for t in range(DSPW):
        strip = wid * DSPW + t
        f = jnp.right_shift(strip, 4)
        d = jnp.bitwise_and(strip, 15)
        row_cp = pltpu.async_copy(tab_t.at[f, d], row_v, rsem)
        if t == 0:
            pltpu.sync_copy(xd_t.at[f], idx_v)
        else:
            prev_f = jnp.right_shift(strip - 1, 4)

            @pl.when(f != prev_f)
            def _():
                pltpu.sync_copy(xd_t.at[f], idx_v)

        if 1 <= t <= CSPW:
            cont_strip(t - 1)
        row_cp.wait()
        for h in range(2):
            if not (t == 0 and h == 0):
                wait_out_write()

            def gk(k, carry):
                ivs = [idx_v[pl.ds(h * HALF + k * 256 + i * 16, 16)]
                       for i in range(16)]
                vals = [plsc.load_gather(row_v, [iv]) for iv in ivs]
                for i in range(16):
                    out_v[pl.ds(k * 256 + i * 16, 16)] = vals[i]
                return carry

            lax.fori_loop(0, HALF // 256, gk, 0, unroll=2)
            pltpu.async_copy(out_v, outh.at[f, d, pl.ds(h * HALF, HALF)], wsem)

    wait_out_write()


_sc_call = pl.kernel(
    _sc_body,
    out_type=jax.ShapeDtypeStruct((N_FIELDS + N_CONT, D, BATCH), jnp.float32),
    mesh=plsc.VectorSubcoreMesh(core_axis_name="c", subcore_axis_name="s"),
    compiler_params=pltpu.CompilerParams(use_tc_tiling_on_sc=True,
                                         needs_layout_passes=False),
    scratch_types=[
        pltpu.VMEM((VOCAB,), jnp.float32),
        pltpu.VMEM((BATCH,), jnp.int32),
        pltpu.VMEM((HALF,), jnp.float32),
        pltpu.VMEM((CONT_STRIPS,), jnp.float32),
        pltpu.SemaphoreType.DMA,
        pltpu.SemaphoreType.DMA,
    ],
)


@jax.jit
def kernel(x_disc, x_cont, tables, cont_w):
    tab_t = tables.transpose(0, 2, 1)
    xd_t = x_disc.astype(jnp.int32).T
    xc_t = x_cont.T
    cwf = cont_w.reshape(CONT_STRIPS)
    out_t = _sc_call(tab_t, xd_t, xc_t, cwf)
    return out_t.transpose(0, 2, 1)
